# Initial kernel scaffold; baseline (speedup 1.0000x reference)
import jax
import jax.numpy as jnp
from jax.experimental import pallas as pl


def kernel(pos, batch):
    raise NotImplementedError("write your pallas kernel here")


# speedup vs baseline: 9.3010x; 9.3010x over previous
import functools

import jax
import jax.numpy as jnp
from jax import lax
from jax.experimental import pallas as pl
from jax.experimental.pallas import tpu as pltpu
from jax.experimental.pallas import tpu_sc as plsc

_CUT2 = 25.0
_K = 32
_N = 4096
_BR = 256


def _topk_body(pos_row_ref, batch_row_ref, post_ref, batch_col_ref,
               src_ref, ew_ref, d2m_ref):
    b = pl.program_id(0)
    px = pos_row_ref[:, 0:1]
    py = pos_row_ref[:, 1:2]
    pz = pos_row_ref[:, 2:3]
    xall = post_ref[0:1, :]
    yall = post_ref[1:2, :]
    zall = post_ref[2:3, :]
    dx = px - xall
    dy = py - yall
    dz = pz - zall
    d2 = dx * dx + dy * dy + dz * dz
    rows = b * _BR + lax.broadcasted_iota(jnp.int32, (_BR, 1), 0)
    cols = lax.broadcasted_iota(jnp.int32, (_BR, _N), 1)
    ok = ((batch_row_ref[:, 0:1] == batch_col_ref[0:1, :])
          & (rows != cols) & (d2 <= _CUT2))
    d2m_ref[...] = jnp.where(ok, d2, 100.0 + cols.astype(jnp.float32))

    kiota = lax.broadcasted_iota(jnp.int32, (1, _K), 1)

    def it(k, _):
        d2m = d2m_ref[...]
        m = jnp.min(d2m, axis=1, keepdims=True)
        idx = jnp.min(jnp.where(d2m == m, cols, _N), axis=1, keepdims=True)
        sel = kiota == k
        src_ref[...] = jnp.where(sel, idx, src_ref[...])
        ew_ref[...] = jnp.where(sel, jnp.where(m < 100.0, jnp.sqrt(m), 0.0),
                                ew_ref[...])
        d2m_ref[...] = jnp.where(cols == idx, jnp.inf, d2m)
        return 0

    lax.fori_loop(0, _K, it, 0)


def _topk(pos, batch):
    posT = pos.T
    batch_row = batch.reshape(_N, 1)
    batch_col = batch.reshape(1, _N)
    grid = (_N // _BR,)
    return pl.pallas_call(
        _topk_body,
        grid=grid,
        in_specs=[
            pl.BlockSpec((_BR, 3), lambda b: (b, 0)),
            pl.BlockSpec((_BR, 1), lambda b: (b, 0)),
            pl.BlockSpec((3, _N), lambda b: (0, 0)),
            pl.BlockSpec((1, _N), lambda b: (0, 0)),
        ],
        out_specs=[
            pl.BlockSpec((_BR, _K), lambda b: (b, 0)),
            pl.BlockSpec((_BR, _K), lambda b: (b, 0)),
        ],
        out_shape=[
            jax.ShapeDtypeStruct((_N, _K), jnp.int32),
            jax.ShapeDtypeStruct((_N, _K), jnp.float32),
        ],
        scratch_shapes=[pltpu.VMEM((_BR, _N), jnp.float32)],
    )(pos, batch_row, posT, batch_col)


_EDGES = _N * _K
_NW = 32
_EPW = _EDGES // _NW
_LANES = 16


_ROWS = 32
_LN = 128


def _edge_gather_body(posx_h, posy_h, posz_h, src_h, dst_h, ew_h,
                      ox_h, oy_h, oz_h,
                      sbuf, dbuf, wbuf, gsx, gsy, gsz, gdx, gdy, gdz,
                      ox, oy, oz, sem):
    wid = lax.axis_index("s") * 2 + lax.axis_index("c")
    pltpu.sync_copy(src_h.at[wid], sbuf)
    pltpu.sync_copy(dst_h.at[wid], dbuf)
    pltpu.sync_copy(ew_h.at[wid], wbuf)

    def fire(j, _):
        pltpu.async_copy(posx_h.at[sbuf.at[j]], gsx.at[j], sem)
        pltpu.async_copy(posy_h.at[sbuf.at[j]], gsy.at[j], sem)
        pltpu.async_copy(posz_h.at[sbuf.at[j]], gsz.at[j], sem)
        pltpu.async_copy(posx_h.at[dbuf.at[j]], gdx.at[j], sem)
        pltpu.async_copy(posy_h.at[dbuf.at[j]], gdy.at[j], sem)
        pltpu.async_copy(posz_h.at[dbuf.at[j]], gdz.at[j], sem)
        return 0

    lax.fori_loop(0, _ROWS, fire, 0)
    for buf in (gsx, gsy, gsz, gdx, gdy, gdz):
        pltpu.make_async_copy(ox_h.at[wid], buf, sem).wait()

    def compute(j, _):
        for t in range(_LN // _LANES):
            sl = pl.ds(t * _LANES, _LANES)
            v = wbuf[j, sl] > 0.0
            ox[j, sl] = jnp.where(v, gsx[j, sl] - gdx[j, sl], 0.0)
            oy[j, sl] = jnp.where(v, gsy[j, sl] - gdy[j, sl], 0.0)
            oz[j, sl] = jnp.where(v, gsz[j, sl] - gdz[j, sl], 0.0)
        return 0

    lax.fori_loop(0, _ROWS, compute, 0)
    pltpu.sync_copy(ox, ox_h.at[wid])
    pltpu.sync_copy(oy, oy_h.at[wid])
    pltpu.sync_copy(oz, oz_h.at[wid])


def _edge_gather(posx, posy, posz, src3, dst3, ew3):
    mesh = plsc.VectorSubcoreMesh(core_axis_name="c", subcore_axis_name="s")
    f = functools.partial(
        pl.kernel,
        mesh=mesh,
        out_type=[jax.ShapeDtypeStruct((_NW, _ROWS, _LN), jnp.float32)] * 3,
        scratch_types=[
            pltpu.VMEM((_ROWS, _LN), jnp.int32),
            pltpu.VMEM((_ROWS, _LN), jnp.int32),
        ] + [pltpu.VMEM((_ROWS, _LN), jnp.float32)] * 10 + [
            pltpu.SemaphoreType.DMA,
        ],
    )(_edge_gather_body)
    return f(posx, posy, posz, src3, dst3, ew3)


def kernel(pos, batch):
    src, ew = _topk(pos, batch)
    src_flat = src.reshape(-1)
    ew_flat = ew.reshape(-1)
    dst = lax.broadcasted_iota(jnp.int32, (_N, _K), 0).reshape(-1)
    evx, evy, evz = _edge_gather(
        pos[:, 0], pos[:, 1], pos[:, 2],
        src_flat.reshape(_NW, _ROWS, _LN),
        dst.reshape(_NW, _ROWS, _LN),
        ew_flat.reshape(_NW, _ROWS, _LN))
    ev = jnp.stack([evx.reshape(-1), evy.reshape(-1), evz.reshape(-1)],
                   axis=1)
    n = _N
    loop_idx = jnp.arange(n, dtype=jnp.int32)
    edge_index = jnp.concatenate(
        [jnp.stack([src_flat, dst]), jnp.stack([loop_idx, loop_idx])], axis=1)
    ew_full = jnp.concatenate([ew_flat, jnp.zeros((n,), jnp.float32)])
    ev_full = jnp.concatenate([ev, jnp.zeros((n, 3), jnp.float32)], axis=0)
    return edge_index, ew_full, ev_full
